# Initial kernel scaffold; baseline (speedup 1.0000x reference)
import jax
import jax.numpy as jnp
from jax.experimental import pallas as pl


def kernel(inputs, edge_index, W_self0, W_neigh0, b0, W_self1, W_neigh1, b1):
    raise NotImplementedError("write your pallas kernel here")


# speedup vs baseline: 3.1486x; 3.1486x over previous
import functools

import jax
import jax.numpy as jnp
from jax import lax
from jax.experimental import pallas as pl
from jax.experimental.pallas import tpu as pltpu
from jax.experimental.pallas import tpu_sc as plsc

N = 10000
N2 = 10240
E = 320000
D = 128
DH = D // 2

NC = 2
NS = 16
E_TILE = E // NS
K = 80
NCHUNK = E_TILE // K
ROWS_TILE = N2 // NS
RCHUNK = 128
NRC = ROWS_TILE // RCHUNK

_mesh = plsc.VectorSubcoreMesh(core_axis_name="c", subcore_axis_name="s")


def _sc_body(with_deg, y_hbm, src_hbm, dst_hbm, *refs):
    if with_deg:
        (agg_out, deg_out, src_v, dst_v, rows_v, ones_v, zbuf, zdeg,
         agg_sh, deg_sh, sem) = refs
    else:
        agg_out, src_v, dst_v, rows_v, zbuf, agg_sh, sem = refs
    c = lax.axis_index("c")
    s = lax.axis_index("s")

    zero16 = jnp.zeros((16,), jnp.float32)

    def _zrow(i, _):
        for j in range(DH // 16):
            zbuf[i, pl.ds(j * 16, 16)] = zero16
        return 0
    lax.fori_loop(0, RCHUNK, _zrow, 0)

    if with_deg:
        one16 = jnp.ones((16,), jnp.float32)

        def _orow(i, _):
            ones_v[i, pl.ds(0, 16)] = one16
            return 0
        lax.fori_loop(0, K, _orow, 0)

        def _zdrow(i, _):
            zdeg[i, pl.ds(0, 16)] = zero16
            return 0
        lax.fori_loop(0, ROWS_TILE, _zdrow, 0)

    rbase = s * ROWS_TILE
    for r in range(NRC):
        pltpu.sync_copy(zbuf, agg_sh.at[pl.ds(rbase + r * RCHUNK, RCHUNK)])
    if with_deg:

        @pl.when(c == 0)
        def _():
            pltpu.sync_copy(zdeg, deg_sh.at[pl.ds(rbase, ROWS_TILE)])
    plsc.subcore_barrier()

    ebase = s * E_TILE
    plane = c * N2

    def _chunk(i, _):
        base = pl.multiple_of(ebase + i * K, 8)
        pltpu.sync_copy(src_hbm.at[pl.ds(base, K)], src_v)
        pltpu.sync_copy(dst_hbm.at[pl.ds(base, K)], dst_v)
        for j in range(K // 16):
            sl = pl.ds(j * 16, 16)
            src_v[sl] = src_v[sl] + plane
        pltpu.async_copy(y_hbm.at[src_v], rows_v, sem).wait()
        pltpu.sync_copy(rows_v, agg_sh.at[dst_v], add=True)
        if with_deg:

            @pl.when(c == 0)
            def _():
                pltpu.sync_copy(ones_v, deg_sh.at[dst_v], add=True)
        return 0
    lax.fori_loop(0, NCHUNK, _chunk, 0)

    plsc.subcore_barrier()

    for r in range(NRC):
        ro = rbase + r * RCHUNK
        pltpu.sync_copy(agg_sh.at[pl.ds(ro, RCHUNK)], zbuf)
        pltpu.sync_copy(zbuf, agg_out.at[c, pl.ds(ro, RCHUNK)])
    if with_deg:

        @pl.when(c == 0)
        def _():
            pltpu.sync_copy(deg_sh.at[pl.ds(rbase, ROWS_TILE)], zdeg)
            pltpu.sync_copy(zdeg, deg_out.at[pl.ds(rbase, ROWS_TILE)])


def _make_sc(with_deg):
    if with_deg:
        out_type = [jax.ShapeDtypeStruct((NC, N2, DH), jnp.float32),
                    jax.ShapeDtypeStruct((N2, 16), jnp.float32)]
        scratch = [
            pltpu.VMEM((K,), jnp.int32),
            pltpu.VMEM((K,), jnp.int32),
            pltpu.VMEM((K, DH), jnp.float32),
            pltpu.VMEM((K, 16), jnp.float32),
            pltpu.VMEM((RCHUNK, DH), jnp.float32),
            pltpu.VMEM((ROWS_TILE, 16), jnp.float32),
            pltpu.VMEM_SHARED((N2, DH), jnp.float32),
            pltpu.VMEM_SHARED((N2, 16), jnp.float32),
            pltpu.SemaphoreType.DMA,
        ]
    else:
        out_type = [jax.ShapeDtypeStruct((NC, N2, DH), jnp.float32)]
        scratch = [
            pltpu.VMEM((K,), jnp.int32),
            pltpu.VMEM((K,), jnp.int32),
            pltpu.VMEM((K, DH), jnp.float32),
            pltpu.VMEM((RCHUNK, DH), jnp.float32),
            pltpu.VMEM_SHARED((N2, DH), jnp.float32),
            pltpu.SemaphoreType.DMA,
        ]
    return pl.kernel(functools.partial(_sc_body, with_deg),
                     out_type=out_type, mesh=_mesh, scratch_types=scratch,
                     compiler_params=pltpu.CompilerParams(
                         use_tc_tiling_on_sc=False),
                     name="sc_scatter_deg" if with_deg else "sc_scatter")


_sc_scatter_deg = _make_sc(True)
_sc_scatter = _make_sc(False)


_GRID = 10
_BN = N2 // _GRID


def _mm_body(x_ref, w_ref, o_ref):
    y = jnp.dot(x_ref[...], w_ref[...], preferred_element_type=jnp.float32)
    o_ref[0] = y[:, :DH]
    o_ref[1] = y[:, DH:]


def _tc_mm(x, w):
    return pl.pallas_call(
        _mm_body,
        grid=(_GRID,),
        in_specs=[pl.BlockSpec((_BN, D), lambda i: (i, 0)),
                  pl.BlockSpec((D, D), lambda i: (0, 0))],
        out_specs=pl.BlockSpec((NC, _BN, DH), lambda i: (0, i, 0)),
        out_shape=jax.ShapeDtypeStruct((NC, N2, DH), jnp.float32),
    )(x, w)


def _fuse_mid_body(x_ref, ws_ref, b_ref, wn_ref, agg_ref, deg_ref,
                   h_ref, y_ref):
    d = jnp.maximum(deg_ref[:, 0:1], 1.0)
    m = jnp.concatenate([agg_ref[0], agg_ref[1]], axis=1) / d
    h = jnp.dot(x_ref[...], ws_ref[...],
                preferred_element_type=jnp.float32) + m + b_ref[...]
    h = jnp.maximum(h, 0.0)
    h_ref[...] = h
    y = jnp.dot(h, wn_ref[...], preferred_element_type=jnp.float32)
    y_ref[0] = y[:, :DH]
    y_ref[1] = y[:, DH:]


def _tc_mid(x, w_self, b, w_neigh, agg, deg):
    return pl.pallas_call(
        _fuse_mid_body,
        grid=(_GRID,),
        in_specs=[pl.BlockSpec((_BN, D), lambda i: (i, 0)),
                  pl.BlockSpec((D, D), lambda i: (0, 0)),
                  pl.BlockSpec((1, D), lambda i: (0, 0)),
                  pl.BlockSpec((D, D), lambda i: (0, 0)),
                  pl.BlockSpec((NC, _BN, DH), lambda i: (0, i, 0)),
                  pl.BlockSpec((_BN, 16), lambda i: (i, 0))],
        out_specs=[pl.BlockSpec((_BN, D), lambda i: (i, 0)),
                   pl.BlockSpec((NC, _BN, DH), lambda i: (0, i, 0))],
        out_shape=[jax.ShapeDtypeStruct((N2, D), jnp.float32),
                   jax.ShapeDtypeStruct((NC, N2, DH), jnp.float32)],
    )(x, w_self, b, w_neigh, agg, deg)


def _fuse_out_body(h_ref, ws_ref, b_ref, agg_ref, deg_ref, o_ref):
    d = jnp.maximum(deg_ref[:, 0:1], 1.0)
    m = jnp.concatenate([agg_ref[0], agg_ref[1]], axis=1) / d
    o_ref[...] = (jnp.dot(h_ref[...], ws_ref[...],
                          preferred_element_type=jnp.float32)
                  + m + b_ref[...])


def _tc_out(h, w_self, b, agg, deg):
    return pl.pallas_call(
        _fuse_out_body,
        grid=(_GRID,),
        in_specs=[pl.BlockSpec((_BN, D), lambda i: (i, 0)),
                  pl.BlockSpec((D, D), lambda i: (0, 0)),
                  pl.BlockSpec((1, D), lambda i: (0, 0)),
                  pl.BlockSpec((NC, _BN, DH), lambda i: (0, i, 0)),
                  pl.BlockSpec((_BN, 16), lambda i: (i, 0))],
        out_specs=pl.BlockSpec((_BN, D), lambda i: (i, 0)),
        out_shape=jax.ShapeDtypeStruct((N2, D), jnp.float32),
    )(h, w_self, b, agg, deg)


def kernel(inputs, edge_index, W_self0, W_neigh0, b0, W_self1, W_neigh1, b1):
    b0r = b0.reshape(1, D)
    b1r = b1.reshape(1, D)
    src = edge_index[0]
    dst = edge_index[1]
    xp = jnp.pad(inputs, ((0, N2 - N), (0, 0)))
    y0 = _tc_mm(xp, W_neigh0).reshape(NC * N2, DH)
    agg0, deg = _sc_scatter_deg(y0, src, dst)
    h1, y1 = _tc_mid(xp, W_self0, b0r, W_neigh1, agg0, deg)
    agg1, = _sc_scatter(y1.reshape(NC * N2, DH), src, dst)
    return _tc_out(h1, W_self1, b1r, agg1, deg)[:N]


# speedup vs baseline: 10.3912x; 3.3003x over previous
import functools

import jax
import jax.numpy as jnp
from jax import lax
from jax.experimental import pallas as pl
from jax.experimental.pallas import tpu as pltpu
from jax.experimental.pallas import tpu_sc as plsc

N = 10000
N2 = 10240
E = 320000
D = 128
DH = D // 2

NC = 2
NS = 16
E_TILE = E // NS
K = 80
NCHUNK = E_TILE // K
ROWS_TILE = N2 // NS
RCHUNK = K
NRC = ROWS_TILE // RCHUNK
G = 10
NSUP = NCHUNK // G
DCHUNK = 128
NDC = ROWS_TILE // DCHUNK

_mesh = plsc.VectorSubcoreMesh(core_axis_name="c", subcore_axis_name="s")


def _sc_body(with_deg, y_hbm, src_hbm, dst_hbm, *refs):
    if with_deg:
        (agg_out, deg_out, src_b, dst_b, rows_v, ones_v, zdeg,
         agg_sh, deg_sh, g0, g1, g2, g3, si0, si1) = refs
    else:
        (agg_out, src_b, dst_b, rows_v, agg_sh,
         g0, g1, g2, g3, si0, si1) = refs
    gsem = (g0, g1, g2, g3)
    isem = (si0, si1)
    c = lax.axis_index("c")
    s = lax.axis_index("s")

    zero16 = jnp.zeros((16,), jnp.float32)

    def _zrow(i, _):
        for b in range(4):
            for j in range(DH // 16):
                rows_v[b, i, pl.ds(j * 16, 16)] = zero16
        return 0
    lax.fori_loop(0, K, _zrow, 0)

    rbase = s * ROWS_TILE
    for r in range(NRC):
        pltpu.sync_copy(rows_v.at[r % 4],
                        agg_sh.at[pl.ds(rbase + r * RCHUNK, RCHUNK)])

    if with_deg:
        one16 = jnp.ones((16,), jnp.float32)

        def _orow(i, _):
            ones_v[i, pl.ds(0, 16)] = one16
            return 0
        lax.fori_loop(0, K, _orow, 0)

        def _zdrow(i, _):
            zdeg[i, pl.ds(0, 16)] = zero16
            return 0
        lax.fori_loop(0, DCHUNK, _zdrow, 0)

        @pl.when(c == 0)
        def _():
            for r in range(NDC):
                pltpu.sync_copy(zdeg, deg_sh.at[pl.ds(rbase + r * DCHUNK,
                                                      DCHUNK)])
    plsc.subcore_barrier()

    cbase = s * NCHUNK
    plane = c * N2

    def _idx_load(u, p):
        pltpu.async_copy(src_hbm.at[pl.ds(cbase + u * G, G)], src_b.at[p],
                         isem[p])
        pltpu.async_copy(dst_hbm.at[pl.ds(cbase + u * G, G)], dst_b.at[p],
                         isem[p])

    def _idx_wait(u, p):
        pltpu.make_async_copy(src_hbm.at[pl.ds(cbase + u * G, G)],
                              src_b.at[p], isem[p]).wait()
        pltpu.make_async_copy(dst_hbm.at[pl.ds(cbase + u * G, G)],
                              dst_b.at[p], isem[p]).wait()

    def _offset(p):
        def _po(r, _):
            for j in range(K // 16):
                sl = pl.ds(j * 16, 16)
                src_b[p, r, sl] = src_b[p, r, sl] + plane
            return 0
        lax.fori_loop(0, G, _po, 0)

    def _gather(p, j, b):
        pltpu.async_copy(y_hbm.at[src_b.at[p, j]], rows_v.at[b], gsem[b])

    def _consume(p, j, b):
        pltpu.make_async_copy(y_hbm.at[src_b.at[p, j]], rows_v.at[b],
                              gsem[b]).wait()
        pltpu.sync_copy(rows_v.at[b], agg_sh.at[dst_b.at[p, j]], add=True)
        if with_deg:

            @pl.when(c == 0)
            def _():
                pltpu.sync_copy(ones_v, deg_sh.at[dst_b.at[p, j]], add=True)

    def _super(u, p, bp):
        for j in range(G - 2):
            _gather(p, j + 2, (bp + j + 2) % 4)
            _consume(p, j, (bp + j) % 4)
        pn = 1 - p
        bn = (bp + G) % 4

        @pl.when(u + 1 < NSUP)
        def _():
            _idx_wait(u + 1, pn)
            _offset(pn)
            _gather(pn, 0, bn)
        _consume(p, G - 2, (bp + G - 2) % 4)

        @pl.when(u + 1 < NSUP)
        def _():
            _gather(pn, 1, (bn + 1) % 4)
        _consume(p, G - 1, (bp + G - 1) % 4)

        @pl.when(u + 2 < NSUP)
        def _():
            _idx_load(u + 2, p)

    _idx_load(0, 0)
    _idx_wait(0, 0)
    _offset(0)
    _idx_load(1, 1)
    _gather(0, 0, 0)
    _gather(0, 1, 1)

    def _pair(t, _):
        u0 = t * 2
        _super(u0, 0, 0)

        @pl.when(u0 + 1 < NSUP)
        def _():
            _super(u0 + 1, 1, 2)
        return 0

    lax.fori_loop(0, (NSUP + 1) // 2, _pair, 0)

    plsc.subcore_barrier()

    for r in range(NRC):
        ro = rbase + r * RCHUNK
        pltpu.sync_copy(agg_sh.at[pl.ds(ro, RCHUNK)], rows_v.at[r % 4])
        pltpu.sync_copy(rows_v.at[r % 4], agg_out.at[c, pl.ds(ro, RCHUNK)])
    if with_deg:

        @pl.when(c == 0)
        def _():
            for r in range(NDC):
                ro = rbase + r * DCHUNK
                pltpu.sync_copy(deg_sh.at[pl.ds(ro, DCHUNK)], zdeg)
                pltpu.sync_copy(zdeg, deg_out.at[pl.ds(ro, DCHUNK)])


def _make_sc(with_deg):
    sems = [pltpu.SemaphoreType.DMA] * 6
    if with_deg:
        out_type = [jax.ShapeDtypeStruct((NC, N2, DH), jnp.float32),
                    jax.ShapeDtypeStruct((N2, 16), jnp.float32)]
        scratch = [
            pltpu.VMEM((2, G, K), jnp.int32),
            pltpu.VMEM((2, G, K), jnp.int32),
            pltpu.VMEM((4, K, DH), jnp.float32),
            pltpu.VMEM((K, 16), jnp.float32),
            pltpu.VMEM((DCHUNK, 16), jnp.float32),
            pltpu.VMEM_SHARED((N2, DH), jnp.float32),
            pltpu.VMEM_SHARED((N2, 16), jnp.float32),
        ] + sems
    else:
        out_type = [jax.ShapeDtypeStruct((NC, N2, DH), jnp.float32)]
        scratch = [
            pltpu.VMEM((2, G, K), jnp.int32),
            pltpu.VMEM((2, G, K), jnp.int32),
            pltpu.VMEM((4, K, DH), jnp.float32),
            pltpu.VMEM_SHARED((N2, DH), jnp.float32),
        ] + sems
    return pl.kernel(functools.partial(_sc_body, with_deg),
                     out_type=out_type, mesh=_mesh, scratch_types=scratch,
                     compiler_params=pltpu.CompilerParams(
                         use_tc_tiling_on_sc=False),
                     name="sc_scatter_deg" if with_deg else "sc_scatter")


_sc_scatter_deg = _make_sc(True)
_sc_scatter = _make_sc(False)


_GRID = 8
_BN = N2 // _GRID


def _mm_body(x_ref, w_ref, o_ref):
    y = jnp.dot(x_ref[...], w_ref[...], preferred_element_type=jnp.float32)
    o_ref[0] = y[:, :DH]
    o_ref[1] = y[:, DH:]


def _tc_mm(x, w):
    return pl.pallas_call(
        _mm_body,
        grid=(_GRID,),
        in_specs=[pl.BlockSpec((_BN, D), lambda i: (i, 0)),
                  pl.BlockSpec((D, D), lambda i: (0, 0))],
        out_specs=pl.BlockSpec((NC, _BN, DH), lambda i: (0, i, 0)),
        out_shape=jax.ShapeDtypeStruct((NC, N2, DH), jnp.float32),
    )(x, w)


def _fuse_mid_body(x_ref, ws_ref, b_ref, wn_ref, agg_ref, deg_ref,
                   h_ref, y_ref):
    d = jnp.maximum(deg_ref[:, 0:1], 1.0)
    m = jnp.concatenate([agg_ref[0], agg_ref[1]], axis=1) / d
    h = jnp.dot(x_ref[...], ws_ref[...],
                preferred_element_type=jnp.float32) + m + b_ref[...]
    h = jnp.maximum(h, 0.0)
    h_ref[...] = h
    y = jnp.dot(h, wn_ref[...], preferred_element_type=jnp.float32)
    y_ref[0] = y[:, :DH]
    y_ref[1] = y[:, DH:]


def _tc_mid(x, w_self, b, w_neigh, agg, deg):
    return pl.pallas_call(
        _fuse_mid_body,
        grid=(_GRID,),
        in_specs=[pl.BlockSpec((_BN, D), lambda i: (i, 0)),
                  pl.BlockSpec((D, D), lambda i: (0, 0)),
                  pl.BlockSpec((1, D), lambda i: (0, 0)),
                  pl.BlockSpec((D, D), lambda i: (0, 0)),
                  pl.BlockSpec((NC, _BN, DH), lambda i: (0, i, 0)),
                  pl.BlockSpec((_BN, 16), lambda i: (i, 0))],
        out_specs=[pl.BlockSpec((_BN, D), lambda i: (i, 0)),
                   pl.BlockSpec((NC, _BN, DH), lambda i: (0, i, 0))],
        out_shape=[jax.ShapeDtypeStruct((N2, D), jnp.float32),
                   jax.ShapeDtypeStruct((NC, N2, DH), jnp.float32)],
    )(x, w_self, b, w_neigh, agg, deg)


def _fuse_out_body(h_ref, ws_ref, b_ref, agg_ref, deg_ref, o_ref):
    d = jnp.maximum(deg_ref[:, 0:1], 1.0)
    m = jnp.concatenate([agg_ref[0], agg_ref[1]], axis=1) / d
    o_ref[...] = (jnp.dot(h_ref[...], ws_ref[...],
                          preferred_element_type=jnp.float32)
                  + m + b_ref[...])


def _tc_out(h, w_self, b, agg, deg):
    return pl.pallas_call(
        _fuse_out_body,
        grid=(_GRID,),
        in_specs=[pl.BlockSpec((_BN, D), lambda i: (i, 0)),
                  pl.BlockSpec((D, D), lambda i: (0, 0)),
                  pl.BlockSpec((1, D), lambda i: (0, 0)),
                  pl.BlockSpec((NC, _BN, DH), lambda i: (0, i, 0)),
                  pl.BlockSpec((_BN, 16), lambda i: (i, 0))],
        out_specs=pl.BlockSpec((_BN, D), lambda i: (i, 0)),
        out_shape=jax.ShapeDtypeStruct((N2, D), jnp.float32),
    )(h, w_self, b, agg, deg)


def kernel(inputs, edge_index, W_self0, W_neigh0, b0, W_self1, W_neigh1, b1):
    b0r = b0.reshape(1, D)
    b1r = b1.reshape(1, D)
    src = edge_index[0].reshape(E // K, K)
    dst = edge_index[1].reshape(E // K, K)
    xp = jnp.pad(inputs, ((0, N2 - N), (0, 0)))
    y0 = _tc_mm(xp, W_neigh0).reshape(NC * N2, DH)
    agg0, deg = _sc_scatter_deg(y0, src, dst)
    h1, y1 = _tc_mid(xp, W_self0, b0r, W_neigh1, agg0, deg)
    agg1, = _sc_scatter(y1.reshape(NC * N2, DH), src, dst)
    return _tc_out(h1, W_self1, b1r, agg1, deg)[:N]


# speedup vs baseline: 10.8113x; 1.0404x over previous
import functools

import jax
import jax.numpy as jnp
from jax import lax
from jax.experimental import pallas as pl
from jax.experimental.pallas import tpu as pltpu
from jax.experimental.pallas import tpu_sc as plsc

N = 10000
N2 = 10240
E = 320000
D = 128
DH = D // 2

NC = 2
NS = 16
E_TILE = E // NS
K = 80
NCHUNK = E_TILE // K
ROWS_TILE = N2 // NS
RCHUNK = K
NRC = ROWS_TILE // RCHUNK
G = 10
NSUP = NCHUNK // G
DCHUNK = 128
NDC = ROWS_TILE // DCHUNK

_mesh = plsc.VectorSubcoreMesh(core_axis_name="c", subcore_axis_name="s")


def _sc_body(with_deg, y_hbm, src_hbm, dst_hbm, *refs):
    if with_deg:
        (agg_out, deg_out, src_b, dst_b, rows_v, ones_v, zdeg,
         agg_sh, deg_sh, g0, g1, g2, g3, si0, si1,
         t0, t1, t2, t3, d0, d1) = refs
    else:
        (agg_out, src_b, dst_b, rows_v, agg_sh,
         g0, g1, g2, g3, si0, si1, t0, t1, t2, t3, d0, d1) = refs
    gsem = (g0, g1, g2, g3)
    isem = (si0, si1)
    ssem = (t0, t1, t2, t3)
    dsem = (d0, d1)
    c = lax.axis_index("c")
    s = lax.axis_index("s")

    zero16 = jnp.zeros((16,), jnp.float32)

    def _zrow(i, _):
        for b in range(4):
            for j in range(DH // 16):
                rows_v[b, i, pl.ds(j * 16, 16)] = zero16
        return 0
    lax.fori_loop(0, K, _zrow, 0)

    rbase = s * ROWS_TILE
    for r in range(NRC):
        pltpu.async_copy(rows_v.at[r % 4],
                         agg_sh.at[pl.ds(rbase + r * RCHUNK, RCHUNK)],
                         gsem[r % 4])
    for r in range(NRC):
        pltpu.make_async_copy(rows_v.at[r % 4],
                              agg_sh.at[pl.ds(rbase + r * RCHUNK, RCHUNK)],
                              gsem[r % 4]).wait()

    if with_deg:
        one16 = jnp.ones((16,), jnp.float32)

        def _orow(i, _):
            ones_v[i, pl.ds(0, 16)] = one16
            return 0
        lax.fori_loop(0, K, _orow, 0)

        def _zdrow(i, _):
            zdeg[i, pl.ds(0, 16)] = zero16
            return 0
        lax.fori_loop(0, DCHUNK, _zdrow, 0)

        @pl.when(c == 0)
        def _():
            for r in range(NDC):
                pltpu.async_copy(zdeg,
                                 deg_sh.at[pl.ds(rbase + r * DCHUNK, DCHUNK)],
                                 dsem[r % 2])
            for r in range(NDC):
                pltpu.make_async_copy(
                    zdeg, deg_sh.at[pl.ds(rbase + r * DCHUNK, DCHUNK)],
                    dsem[r % 2]).wait()
    plsc.subcore_barrier()

    cbase = s * NCHUNK
    plane = c * N2

    def _idx_load(u, p):
        pltpu.async_copy(src_hbm.at[pl.ds(cbase + u * G, G)], src_b.at[p],
                         isem[p])
        pltpu.async_copy(dst_hbm.at[pl.ds(cbase + u * G, G)], dst_b.at[p],
                         isem[p])

    def _idx_wait(u, p):
        pltpu.make_async_copy(src_hbm.at[pl.ds(cbase + u * G, G)],
                              src_b.at[p], isem[p]).wait()
        pltpu.make_async_copy(dst_hbm.at[pl.ds(cbase + u * G, G)],
                              dst_b.at[p], isem[p]).wait()

    def _offset(p):
        def _po(r, _):
            for j in range(K // 16):
                sl = pl.ds(j * 16, 16)
                src_b[p, r, sl] = src_b[p, r, sl] + plane
            return 0
        lax.fori_loop(0, G, _po, 0)

    def _gather(p, j, b):
        pltpu.async_copy(y_hbm.at[src_b.at[p, j]], rows_v.at[b], gsem[b])

    def _swait(b):
        pltpu.make_async_copy(rows_v.at[b], agg_sh.at[dst_b.at[0, 0]],
                              ssem[b]).wait()

    def _dwait(q):
        pltpu.make_async_copy(ones_v, deg_sh.at[dst_b.at[0, 0]],
                              dsem[q]).wait()

    def _consume(p, j, b, deg_wait):
        pltpu.make_async_copy(y_hbm.at[src_b.at[p, j]], rows_v.at[b],
                              gsem[b]).wait()
        pltpu.async_copy(rows_v.at[b], agg_sh.at[dst_b.at[p, j]], ssem[b],
                         add=True)
        if with_deg:

            @pl.when(c == 0)
            def _():
                if deg_wait:
                    _dwait(j % 2)
                pltpu.async_copy(ones_v, deg_sh.at[dst_b.at[p, j]],
                                 dsem[j % 2], add=True)

    def _super(u, p, bp):
        for j in range(G - 2):
            if j >= 2:
                _swait((bp + j - 2) % 4)
            _gather(p, j + 2, (bp + j + 2) % 4)
            _consume(p, j, (bp + j) % 4, deg_wait=(j >= 2))
        pn = 1 - p
        bn = (bp + 2) % 4

        @pl.when(u + 1 < NSUP)
        def _():
            _swait((bp + 2) % 4)
            _idx_wait(u + 1, pn)
            _offset(pn)
            _gather(pn, 0, bn)
        _consume(p, G - 2, bp, deg_wait=True)

        @pl.when(u + 1 < NSUP)
        def _():
            _swait((bp + 3) % 4)
            _gather(pn, 1, (bn + 1) % 4)
        _consume(p, G - 1, (bp + 1) % 4, deg_wait=True)

        _swait(bp)
        _swait((bp + 1) % 4)
        if with_deg:

            @pl.when(c == 0)
            def _():
                _dwait(0)
                _dwait(1)

        @pl.when(u + 2 < NSUP)
        def _():
            _idx_load(u + 2, p)

    _idx_load(0, 0)
    _idx_wait(0, 0)
    _offset(0)
    _idx_load(1, 1)
    _gather(0, 0, 0)
    _gather(0, 1, 1)

    def _pair(t, _):
        u0 = t * 2
        _super(u0, 0, 0)

        @pl.when(u0 + 1 < NSUP)
        def _():
            _super(u0 + 1, 1, 2)
        return 0

    lax.fori_loop(0, (NSUP + 1) // 2, _pair, 0)

    _swait(2)
    _swait(3)
    plsc.subcore_barrier()

    for r in range(NRC):
        ro = rbase + r * RCHUNK
        pltpu.sync_copy(agg_sh.at[pl.ds(ro, RCHUNK)], rows_v.at[r % 4])
        pltpu.sync_copy(rows_v.at[r % 4], agg_out.at[c, pl.ds(ro, RCHUNK)])
    if with_deg:

        @pl.when(c == 0)
        def _():
            for r in range(NDC):
                ro = rbase + r * DCHUNK
                pltpu.sync_copy(deg_sh.at[pl.ds(ro, DCHUNK)], zdeg)
                pltpu.sync_copy(zdeg, deg_out.at[pl.ds(ro, DCHUNK)])


def _make_sc(with_deg):
    sems = [pltpu.SemaphoreType.DMA] * 12
    if with_deg:
        out_type = [jax.ShapeDtypeStruct((NC, N2, DH), jnp.float32),
                    jax.ShapeDtypeStruct((N2, 16), jnp.float32)]
        scratch = [
            pltpu.VMEM((2, G, K), jnp.int32),
            pltpu.VMEM((2, G, K), jnp.int32),
            pltpu.VMEM((4, K, DH), jnp.float32),
            pltpu.VMEM((K, 16), jnp.float32),
            pltpu.VMEM((DCHUNK, 16), jnp.float32),
            pltpu.VMEM_SHARED((N2, DH), jnp.float32),
            pltpu.VMEM_SHARED((N2, 16), jnp.float32),
        ] + sems
    else:
        out_type = [jax.ShapeDtypeStruct((NC, N2, DH), jnp.float32)]
        scratch = [
            pltpu.VMEM((2, G, K), jnp.int32),
            pltpu.VMEM((2, G, K), jnp.int32),
            pltpu.VMEM((4, K, DH), jnp.float32),
            pltpu.VMEM_SHARED((N2, DH), jnp.float32),
        ] + sems
    return pl.kernel(functools.partial(_sc_body, with_deg),
                     out_type=out_type, mesh=_mesh, scratch_types=scratch,
                     compiler_params=pltpu.CompilerParams(
                         use_tc_tiling_on_sc=False),
                     name="sc_scatter_deg" if with_deg else "sc_scatter")


_sc_scatter_deg = _make_sc(True)
_sc_scatter = _make_sc(False)


_GRID = 10
_BN = 1024


def _mm_body(x_ref, w_ref, o_ref):
    y = jnp.dot(x_ref[...], w_ref[...], preferred_element_type=jnp.float32)
    o_ref[0] = y[:, :DH]
    o_ref[1] = y[:, DH:]


def _tc_mm(x, w):
    return pl.pallas_call(
        _mm_body,
        grid=(_GRID,),
        in_specs=[pl.BlockSpec((_BN, D), lambda i: (i, 0)),
                  pl.BlockSpec((D, D), lambda i: (0, 0))],
        out_specs=pl.BlockSpec((NC, _BN, DH), lambda i: (0, i, 0)),
        out_shape=jax.ShapeDtypeStruct((NC, N2, DH), jnp.float32),
    )(x, w)


def _fuse_mid_body(x_ref, ws_ref, b_ref, wn_ref, agg_ref, deg_ref,
                   h_ref, y_ref):
    d = jnp.maximum(deg_ref[:, 0:1], 1.0)
    m = jnp.concatenate([agg_ref[0], agg_ref[1]], axis=1) / d
    h = jnp.dot(x_ref[...], ws_ref[...],
                preferred_element_type=jnp.float32) + m + b_ref[...]
    h = jnp.maximum(h, 0.0)
    h_ref[...] = h
    y = jnp.dot(h, wn_ref[...], preferred_element_type=jnp.float32)
    y_ref[0] = y[:, :DH]
    y_ref[1] = y[:, DH:]


def _tc_mid(x, w_self, b, w_neigh, agg, deg):
    return pl.pallas_call(
        _fuse_mid_body,
        grid=(_GRID,),
        in_specs=[pl.BlockSpec((_BN, D), lambda i: (i, 0)),
                  pl.BlockSpec((D, D), lambda i: (0, 0)),
                  pl.BlockSpec((1, D), lambda i: (0, 0)),
                  pl.BlockSpec((D, D), lambda i: (0, 0)),
                  pl.BlockSpec((NC, _BN, DH), lambda i: (0, i, 0)),
                  pl.BlockSpec((_BN, 16), lambda i: (i, 0))],
        out_specs=[pl.BlockSpec((_BN, D), lambda i: (i, 0)),
                   pl.BlockSpec((NC, _BN, DH), lambda i: (0, i, 0))],
        out_shape=[jax.ShapeDtypeStruct((N, D), jnp.float32),
                   jax.ShapeDtypeStruct((NC, N2, DH), jnp.float32)],
    )(x, w_self, b, w_neigh, agg, deg)


def _fuse_out_body(h_ref, ws_ref, b_ref, agg_ref, deg_ref, o_ref):
    d = jnp.maximum(deg_ref[:, 0:1], 1.0)
    m = jnp.concatenate([agg_ref[0], agg_ref[1]], axis=1) / d
    o_ref[...] = (jnp.dot(h_ref[...], ws_ref[...],
                          preferred_element_type=jnp.float32)
                  + m + b_ref[...])


def _tc_out(h, w_self, b, agg, deg):
    return pl.pallas_call(
        _fuse_out_body,
        grid=(_GRID,),
        in_specs=[pl.BlockSpec((_BN, D), lambda i: (i, 0)),
                  pl.BlockSpec((D, D), lambda i: (0, 0)),
                  pl.BlockSpec((1, D), lambda i: (0, 0)),
                  pl.BlockSpec((NC, _BN, DH), lambda i: (0, i, 0)),
                  pl.BlockSpec((_BN, 16), lambda i: (i, 0))],
        out_specs=pl.BlockSpec((_BN, D), lambda i: (i, 0)),
        out_shape=jax.ShapeDtypeStruct((N, D), jnp.float32),
    )(h, w_self, b, agg, deg)


def kernel(inputs, edge_index, W_self0, W_neigh0, b0, W_self1, W_neigh1, b1):
    b0r = b0.reshape(1, D)
    b1r = b1.reshape(1, D)
    src = edge_index[0].reshape(E // K, K)
    dst = edge_index[1].reshape(E // K, K)
    y0 = _tc_mm(inputs, W_neigh0).reshape(NC * N2, DH)
    agg0, deg = _sc_scatter_deg(y0, src, dst)
    h1, y1 = _tc_mid(inputs, W_self0, b0r, W_neigh1, agg0, deg)
    agg1, = _sc_scatter(y1.reshape(NC * N2, DH), src, dst)
    return _tc_out(h1, W_self1, b1r, agg1, deg)


# speedup vs baseline: 11.1801x; 1.0341x over previous
import functools

import jax
import jax.numpy as jnp
from jax import lax
from jax.experimental import pallas as pl
from jax.experimental.pallas import tpu as pltpu
from jax.experimental.pallas import tpu_sc as plsc

N = 10000
N2 = 10240
E = 320000
D = 128
DH = D // 2

NC = 2
NS = 16
E_TILE = E // NS
K = 80
NCHUNK = E_TILE // K
ROWS_TILE = N2 // NS
RCHUNK = K
NRC = ROWS_TILE // RCHUNK
G = 10
NSUP = NCHUNK // G
DCHUNK = 128
NDC = ROWS_TILE // DCHUNK

_mesh = plsc.VectorSubcoreMesh(core_axis_name="c", subcore_axis_name="s")


def _sc_body(with_deg, y_hbm, src_hbm, dst_hbm, *refs):
    if with_deg:
        (agg_out, deg_out, src_b, dst_b, rows_v, ones_v, zdeg,
         agg_sh, deg_sh, g0, g1, g2, g3, si0, si1,
         t0, t1, t2, t3, d0, d1) = refs
    else:
        (agg_out, src_b, dst_b, rows_v, agg_sh,
         g0, g1, g2, g3, si0, si1, t0, t1, t2, t3, d0, d1) = refs
    gsem = (g0, g1, g2, g3)
    isem = (si0, si1)
    ssem = (t0, t1, t2, t3)
    dsem = (d0, d1)
    c = lax.axis_index("c")
    s = lax.axis_index("s")

    zero16 = jnp.zeros((16,), jnp.float32)

    def _zrow(i, _):
        for b in range(4):
            for j in range(DH // 16):
                rows_v[b, i, pl.ds(j * 16, 16)] = zero16
        return 0
    lax.fori_loop(0, K, _zrow, 0)

    rbase = s * ROWS_TILE
    for r in range(NRC):
        pltpu.async_copy(rows_v.at[r % 4],
                         agg_sh.at[pl.ds(rbase + r * RCHUNK, RCHUNK)],
                         gsem[r % 4])
    for r in range(NRC):
        pltpu.make_async_copy(rows_v.at[r % 4],
                              agg_sh.at[pl.ds(rbase + r * RCHUNK, RCHUNK)],
                              gsem[r % 4]).wait()

    if with_deg:
        one16 = jnp.ones((16,), jnp.float32)

        def _orow(i, _):
            ones_v[i, pl.ds(0, 16)] = one16
            return 0
        lax.fori_loop(0, K, _orow, 0)

        def _zdrow(i, _):
            zdeg[i, pl.ds(0, 16)] = zero16
            return 0
        lax.fori_loop(0, DCHUNK, _zdrow, 0)

        for r in range(NDC):
            pltpu.async_copy(zdeg,
                             deg_sh.at[pl.ds(rbase + r * DCHUNK, DCHUNK)],
                             dsem[r % 2])
        for r in range(NDC):
            pltpu.make_async_copy(
                zdeg, deg_sh.at[pl.ds(rbase + r * DCHUNK, DCHUNK)],
                dsem[r % 2]).wait()
    plsc.subcore_barrier()

    cbase = s * NCHUNK
    plane = c * N2

    def _idx_load(u, p):
        pltpu.async_copy(src_hbm.at[pl.ds(cbase + u * G, G)], src_b.at[p],
                         isem[p])
        pltpu.async_copy(dst_hbm.at[pl.ds(cbase + u * G, G)], dst_b.at[p],
                         isem[p])

    def _idx_wait(u, p):
        pltpu.make_async_copy(src_hbm.at[pl.ds(cbase + u * G, G)],
                              src_b.at[p], isem[p]).wait()
        pltpu.make_async_copy(dst_hbm.at[pl.ds(cbase + u * G, G)],
                              dst_b.at[p], isem[p]).wait()

    def _offset(p):
        def _po(r, _):
            for j in range(K // 16):
                sl = pl.ds(j * 16, 16)
                src_b[p, r, sl] = src_b[p, r, sl] + plane
            return 0
        lax.fori_loop(0, G, _po, 0)

    def _gather(p, j, b):
        pltpu.async_copy(y_hbm.at[src_b.at[p, j]], rows_v.at[b], gsem[b])

    def _swait(b):
        pltpu.make_async_copy(rows_v.at[b], agg_sh.at[dst_b.at[0, 0]],
                              ssem[b]).wait()

    def _dwait(q):
        pltpu.make_async_copy(ones_v, deg_sh.at[dst_b.at[0, 0]],
                              dsem[q]).wait()

    def _consume(p, j, b, deg_wait, degc):
        pltpu.make_async_copy(y_hbm.at[src_b.at[p, j]], rows_v.at[b],
                              gsem[b]).wait()
        pltpu.async_copy(rows_v.at[b], agg_sh.at[dst_b.at[p, j]], ssem[b],
                         add=True)
        if with_deg:

            @pl.when(c == degc)
            def _():
                if deg_wait:
                    _dwait(j % 2)
                pltpu.async_copy(ones_v, deg_sh.at[dst_b.at[p, j]],
                                 dsem[j % 2], add=True)

    def _super(u, p, bp, degc):
        for j in range(G - 2):
            if j >= 2:
                _swait((bp + j - 2) % 4)
            _gather(p, j + 2, (bp + j + 2) % 4)
            _consume(p, j, (bp + j) % 4, deg_wait=(j >= 2), degc=degc)
        pn = 1 - p
        bn = (bp + 2) % 4

        @pl.when(u + 1 < NSUP)
        def _():
            _swait((bp + 2) % 4)
            _idx_wait(u + 1, pn)
            _offset(pn)
            _gather(pn, 0, bn)
        _consume(p, G - 2, bp, deg_wait=True, degc=degc)

        @pl.when(u + 1 < NSUP)
        def _():
            _swait((bp + 3) % 4)
            _gather(pn, 1, (bn + 1) % 4)
        _consume(p, G - 1, (bp + 1) % 4, deg_wait=True, degc=degc)

        _swait(bp)
        _swait((bp + 1) % 4)
        if with_deg:

            @pl.when(c == degc)
            def _():
                _dwait(0)
                _dwait(1)

        @pl.when(u + 2 < NSUP)
        def _():
            _idx_load(u + 2, p)

    _idx_load(0, 0)
    _idx_wait(0, 0)
    _offset(0)
    _idx_load(1, 1)
    _gather(0, 0, 0)
    _gather(0, 1, 1)

    def _pair(t, _):
        u0 = t * 2
        _super(u0, 0, 0, degc=0)

        @pl.when(u0 + 1 < NSUP)
        def _():
            _super(u0 + 1, 1, 2, degc=1)
        return 0

    lax.fori_loop(0, (NSUP + 1) // 2, _pair, 0)

    _swait(2)
    _swait(3)
    plsc.subcore_barrier()

    for r in range(NRC):
        ro = rbase + r * RCHUNK
        pltpu.sync_copy(agg_sh.at[pl.ds(ro, RCHUNK)],
                        agg_out.at[c, pl.ds(ro, RCHUNK)])
    if with_deg:
        pltpu.sync_copy(deg_sh.at[pl.ds(rbase, ROWS_TILE)],
                        deg_out.at[c, pl.ds(rbase, ROWS_TILE)])


def _make_sc(with_deg):
    sems = [pltpu.SemaphoreType.DMA] * 12
    if with_deg:
        out_type = [jax.ShapeDtypeStruct((NC, N2, DH), jnp.float32),
                    jax.ShapeDtypeStruct((NC, N2, 16), jnp.float32)]
        scratch = [
            pltpu.VMEM((2, G, K), jnp.int32),
            pltpu.VMEM((2, G, K), jnp.int32),
            pltpu.VMEM((4, K, DH), jnp.float32),
            pltpu.VMEM((K, 16), jnp.float32),
            pltpu.VMEM((DCHUNK, 16), jnp.float32),
            pltpu.VMEM_SHARED((N2, DH), jnp.float32),
            pltpu.VMEM_SHARED((N2, 16), jnp.float32),
        ] + sems
    else:
        out_type = [jax.ShapeDtypeStruct((NC, N2, DH), jnp.float32)]
        scratch = [
            pltpu.VMEM((2, G, K), jnp.int32),
            pltpu.VMEM((2, G, K), jnp.int32),
            pltpu.VMEM((4, K, DH), jnp.float32),
            pltpu.VMEM_SHARED((N2, DH), jnp.float32),
        ] + sems
    return pl.kernel(functools.partial(_sc_body, with_deg),
                     out_type=out_type, mesh=_mesh, scratch_types=scratch,
                     compiler_params=pltpu.CompilerParams(
                         use_tc_tiling_on_sc=False),
                     name="sc_scatter_deg" if with_deg else "sc_scatter")


_sc_scatter_deg = _make_sc(True)
_sc_scatter = _make_sc(False)


_GRID = 10
_BN = 1024


def _mm_body(x_ref, w_ref, o_ref):
    y = jnp.dot(x_ref[...], w_ref[...], preferred_element_type=jnp.float32)
    o_ref[0] = y[:, :DH]
    o_ref[1] = y[:, DH:]


def _tc_mm(x, w):
    return pl.pallas_call(
        _mm_body,
        grid=(_GRID,),
        in_specs=[pl.BlockSpec((_BN, D), lambda i: (i, 0)),
                  pl.BlockSpec((D, D), lambda i: (0, 0))],
        out_specs=pl.BlockSpec((NC, _BN, DH), lambda i: (0, i, 0)),
        out_shape=jax.ShapeDtypeStruct((NC, N2, DH), jnp.float32),
    )(x, w)


def _fuse_mid_body(x_ref, ws_ref, b_ref, wn_ref, agg_ref, deg_ref,
                   h_ref, y_ref):
    d = jnp.maximum(deg_ref[0][:, 0:1] + deg_ref[1][:, 0:1], 1.0)
    m = jnp.concatenate([agg_ref[0], agg_ref[1]], axis=1) / d
    h = jnp.dot(x_ref[...], ws_ref[...],
                preferred_element_type=jnp.float32) + m + b_ref[...]
    h = jnp.maximum(h, 0.0)
    h_ref[...] = h
    y = jnp.dot(h, wn_ref[...], preferred_element_type=jnp.float32)
    y_ref[0] = y[:, :DH]
    y_ref[1] = y[:, DH:]


def _tc_mid(x, w_self, b, w_neigh, agg, deg):
    return pl.pallas_call(
        _fuse_mid_body,
        grid=(_GRID,),
        in_specs=[pl.BlockSpec((_BN, D), lambda i: (i, 0)),
                  pl.BlockSpec((D, D), lambda i: (0, 0)),
                  pl.BlockSpec((1, D), lambda i: (0, 0)),
                  pl.BlockSpec((D, D), lambda i: (0, 0)),
                  pl.BlockSpec((NC, _BN, DH), lambda i: (0, i, 0)),
                  pl.BlockSpec((NC, _BN, 16), lambda i: (0, i, 0))],
        out_specs=[pl.BlockSpec((_BN, D), lambda i: (i, 0)),
                   pl.BlockSpec((NC, _BN, DH), lambda i: (0, i, 0))],
        out_shape=[jax.ShapeDtypeStruct((N, D), jnp.float32),
                   jax.ShapeDtypeStruct((NC, N2, DH), jnp.float32)],
    )(x, w_self, b, w_neigh, agg, deg)


def _fuse_out_body(h_ref, ws_ref, b_ref, agg_ref, deg_ref, o_ref):
    d = jnp.maximum(deg_ref[0][:, 0:1] + deg_ref[1][:, 0:1], 1.0)
    m = jnp.concatenate([agg_ref[0], agg_ref[1]], axis=1) / d
    o_ref[...] = (jnp.dot(h_ref[...], ws_ref[...],
                          preferred_element_type=jnp.float32)
                  + m + b_ref[...])


def _tc_out(h, w_self, b, agg, deg):
    return pl.pallas_call(
        _fuse_out_body,
        grid=(_GRID,),
        in_specs=[pl.BlockSpec((_BN, D), lambda i: (i, 0)),
                  pl.BlockSpec((D, D), lambda i: (0, 0)),
                  pl.BlockSpec((1, D), lambda i: (0, 0)),
                  pl.BlockSpec((NC, _BN, DH), lambda i: (0, i, 0)),
                  pl.BlockSpec((NC, _BN, 16), lambda i: (0, i, 0))],
        out_specs=pl.BlockSpec((_BN, D), lambda i: (i, 0)),
        out_shape=jax.ShapeDtypeStruct((N, D), jnp.float32),
    )(h, w_self, b, agg, deg)


def kernel(inputs, edge_index, W_self0, W_neigh0, b0, W_self1, W_neigh1, b1):
    b0r = b0.reshape(1, D)
    b1r = b1.reshape(1, D)
    src = edge_index[0].reshape(E // K, K)
    dst = edge_index[1].reshape(E // K, K)
    y0 = _tc_mm(inputs, W_neigh0).reshape(NC * N2, DH)
    agg0, deg = _sc_scatter_deg(y0, src, dst)
    h1, y1 = _tc_mid(inputs, W_self0, b0r, W_neigh1, agg0, deg)
    agg1, = _sc_scatter(y1.reshape(NC * N2, DH), src, dst)
    return _tc_out(h1, W_self1, b1r, agg1, deg)


# speedup vs baseline: 11.1864x; 1.0006x over previous
import functools

import jax
import jax.numpy as jnp
from jax import lax
from jax.experimental import pallas as pl
from jax.experimental.pallas import tpu as pltpu
from jax.experimental.pallas import tpu_sc as plsc

N = 10000
N2 = 10240
E = 320000
D = 128
DH = D // 2

NC = 2
NS = 16
E_TILE = E // NS
K = 80
NCHUNK = E_TILE // K
ROWS_TILE = N2 // NS
RCHUNK = K
NRC = ROWS_TILE // RCHUNK
G = 10
NSUP = NCHUNK // G
DCHUNK = 128
NDC = ROWS_TILE // DCHUNK

_mesh = plsc.VectorSubcoreMesh(core_axis_name="c", subcore_axis_name="s")


def _sc_body(with_deg, y_hbm, edge_hbm, *refs):
    if with_deg:
        (agg_out, deg_out, src_b, dst_b, rows_v, ones_v, zdeg,
         agg_sh, deg_sh, g0, g1, g2, g3, si0, si1,
         t0, t1, t2, t3, d0, d1) = refs
    else:
        (agg_out, src_b, dst_b, rows_v, agg_sh,
         g0, g1, g2, g3, si0, si1, t0, t1, t2, t3, d0, d1) = refs
    gsem = (g0, g1, g2, g3)
    isem = (si0, si1)
    ssem = (t0, t1, t2, t3)
    dsem = (d0, d1)
    c = lax.axis_index("c")
    s = lax.axis_index("s")

    zero16 = jnp.zeros((16,), jnp.float32)

    def _zrow(i, _):
        for b in range(4):
            for j in range(DH // 16):
                rows_v[b, i, pl.ds(j * 16, 16)] = zero16
        return 0
    lax.fori_loop(0, K, _zrow, 0)

    rbase = s * ROWS_TILE
    for r in range(NRC):
        pltpu.async_copy(rows_v.at[r % 4],
                         agg_sh.at[pl.ds(rbase + r * RCHUNK, RCHUNK)],
                         gsem[r % 4])
    for r in range(NRC):
        pltpu.make_async_copy(rows_v.at[r % 4],
                              agg_sh.at[pl.ds(rbase + r * RCHUNK, RCHUNK)],
                              gsem[r % 4]).wait()

    if with_deg:
        one16 = jnp.ones((16,), jnp.float32)

        def _orow(i, _):
            ones_v[i, pl.ds(0, 16)] = one16
            return 0
        lax.fori_loop(0, K, _orow, 0)

        def _zdrow(i, _):
            zdeg[i, pl.ds(0, 16)] = zero16
            return 0
        lax.fori_loop(0, DCHUNK, _zdrow, 0)

        for r in range(NDC):
            pltpu.async_copy(zdeg,
                             deg_sh.at[pl.ds(rbase + r * DCHUNK, DCHUNK)],
                             dsem[r % 2])
        for r in range(NDC):
            pltpu.make_async_copy(
                zdeg, deg_sh.at[pl.ds(rbase + r * DCHUNK, DCHUNK)],
                dsem[r % 2]).wait()
    plsc.subcore_barrier()

    cbase = s * NCHUNK

    def _idx_load(u, p):
        pltpu.async_copy(edge_hbm.at[0, pl.ds(cbase + u * G, G)], src_b.at[p],
                         isem[p])
        pltpu.async_copy(edge_hbm.at[1, pl.ds(cbase + u * G, G)], dst_b.at[p],
                         isem[p])

    def _idx_wait(u, p):
        pltpu.make_async_copy(edge_hbm.at[0, pl.ds(cbase + u * G, G)],
                              src_b.at[p], isem[p]).wait()
        pltpu.make_async_copy(edge_hbm.at[1, pl.ds(cbase + u * G, G)],
                              dst_b.at[p], isem[p]).wait()

    def _gather(p, j, b):
        pltpu.async_copy(y_hbm.at[c].at[src_b.at[p, j]], rows_v.at[b],
                         gsem[b])

    def _swait(b):
        pltpu.make_async_copy(rows_v.at[b], agg_sh.at[dst_b.at[0, 0]],
                              ssem[b]).wait()

    def _dwait(q):
        pltpu.make_async_copy(ones_v, deg_sh.at[dst_b.at[0, 0]],
                              dsem[q]).wait()

    def _consume(p, j, b, deg_wait, degc):
        pltpu.make_async_copy(y_hbm.at[c].at[src_b.at[p, j]], rows_v.at[b],
                              gsem[b]).wait()
        pltpu.async_copy(rows_v.at[b], agg_sh.at[dst_b.at[p, j]], ssem[b],
                         add=True)
        if with_deg:

            @pl.when(c == degc)
            def _():
                if deg_wait:
                    _dwait(j % 2)
                pltpu.async_copy(ones_v, deg_sh.at[dst_b.at[p, j]],
                                 dsem[j % 2], add=True)

    def _super(u, p, bp, degc):
        for j in range(G - 2):
            if j >= 2:
                _swait((bp + j - 2) % 4)
            _gather(p, j + 2, (bp + j + 2) % 4)
            _consume(p, j, (bp + j) % 4, deg_wait=(j >= 2), degc=degc)
        pn = 1 - p
        bn = (bp + 2) % 4

        @pl.when(u + 1 < NSUP)
        def _():
            _swait((bp + 2) % 4)
            _idx_wait(u + 1, pn)
            _gather(pn, 0, bn)
        _consume(p, G - 2, bp, deg_wait=True, degc=degc)

        @pl.when(u + 1 < NSUP)
        def _():
            _swait((bp + 3) % 4)
            _gather(pn, 1, (bn + 1) % 4)
        _consume(p, G - 1, (bp + 1) % 4, deg_wait=True, degc=degc)

        _swait(bp)
        _swait((bp + 1) % 4)
        if with_deg:

            @pl.when(c == degc)
            def _():
                _dwait(0)
                _dwait(1)

        @pl.when(u + 2 < NSUP)
        def _():
            _idx_load(u + 2, p)

    _idx_load(0, 0)
    _idx_wait(0, 0)
    _idx_load(1, 1)
    _gather(0, 0, 0)
    _gather(0, 1, 1)

    def _pair(t, _):
        u0 = t * 2
        _super(u0, 0, 0, degc=0)

        @pl.when(u0 + 1 < NSUP)
        def _():
            _super(u0 + 1, 1, 2, degc=1)
        return 0

    lax.fori_loop(0, (NSUP + 1) // 2, _pair, 0)

    _swait(2)
    _swait(3)
    plsc.subcore_barrier()

    for r in range(NRC):
        ro = rbase + r * RCHUNK
        pltpu.sync_copy(agg_sh.at[pl.ds(ro, RCHUNK)],
                        agg_out.at[c, pl.ds(ro, RCHUNK)])
    if with_deg:
        pltpu.sync_copy(deg_sh.at[pl.ds(rbase, ROWS_TILE)],
                        deg_out.at[c, pl.ds(rbase, ROWS_TILE)])


def _make_sc(with_deg):
    sems = [pltpu.SemaphoreType.DMA] * 12
    if with_deg:
        out_type = [jax.ShapeDtypeStruct((NC, N2, DH), jnp.float32),
                    jax.ShapeDtypeStruct((NC, N2, 16), jnp.float32)]
        scratch = [
            pltpu.VMEM((2, G, K), jnp.int32),
            pltpu.VMEM((2, G, K), jnp.int32),
            pltpu.VMEM((4, K, DH), jnp.float32),
            pltpu.VMEM((K, 16), jnp.float32),
            pltpu.VMEM((DCHUNK, 16), jnp.float32),
            pltpu.VMEM_SHARED((N2, DH), jnp.float32),
            pltpu.VMEM_SHARED((N2, 16), jnp.float32),
        ] + sems
    else:
        out_type = [jax.ShapeDtypeStruct((NC, N2, DH), jnp.float32)]
        scratch = [
            pltpu.VMEM((2, G, K), jnp.int32),
            pltpu.VMEM((2, G, K), jnp.int32),
            pltpu.VMEM((4, K, DH), jnp.float32),
            pltpu.VMEM_SHARED((N2, DH), jnp.float32),
        ] + sems
    return pl.kernel(functools.partial(_sc_body, with_deg),
                     out_type=out_type, mesh=_mesh, scratch_types=scratch,
                     compiler_params=pltpu.CompilerParams(
                         use_tc_tiling_on_sc=False),
                     name="sc_scatter_deg" if with_deg else "sc_scatter")


_sc_scatter_deg = _make_sc(True)
_sc_scatter = _make_sc(False)


_GRID = 10
_BN = 1024


def _mm_body(x_ref, w_ref, o_ref):
    y = jnp.dot(x_ref[...], w_ref[...], preferred_element_type=jnp.float32)
    o_ref[0] = y[:, :DH]
    o_ref[1] = y[:, DH:]


def _tc_mm(x, w):
    return pl.pallas_call(
        _mm_body,
        grid=(_GRID,),
        in_specs=[pl.BlockSpec((_BN, D), lambda i: (i, 0)),
                  pl.BlockSpec((D, D), lambda i: (0, 0))],
        out_specs=pl.BlockSpec((NC, _BN, DH), lambda i: (0, i, 0)),
        out_shape=jax.ShapeDtypeStruct((NC, N2, DH), jnp.float32),
    )(x, w)


def _fuse_mid_body(x_ref, ws_ref, b_ref, wn_ref, agg_ref, deg_ref,
                   h_ref, y_ref):
    d = jnp.maximum(deg_ref[0][:, 0:1] + deg_ref[1][:, 0:1], 1.0)
    m = jnp.concatenate([agg_ref[0], agg_ref[1]], axis=1) / d
    h = jnp.dot(x_ref[...], ws_ref[...],
                preferred_element_type=jnp.float32) + m + b_ref[...]
    h = jnp.maximum(h, 0.0)
    h_ref[...] = h
    y = jnp.dot(h, wn_ref[...], preferred_element_type=jnp.float32)
    y_ref[0] = y[:, :DH]
    y_ref[1] = y[:, DH:]


def _tc_mid(x, w_self, b, w_neigh, agg, deg):
    return pl.pallas_call(
        _fuse_mid_body,
        grid=(_GRID,),
        in_specs=[pl.BlockSpec((_BN, D), lambda i: (i, 0)),
                  pl.BlockSpec((D, D), lambda i: (0, 0)),
                  pl.BlockSpec((1, D), lambda i: (0, 0)),
                  pl.BlockSpec((D, D), lambda i: (0, 0)),
                  pl.BlockSpec((NC, _BN, DH), lambda i: (0, i, 0)),
                  pl.BlockSpec((NC, _BN, 16), lambda i: (0, i, 0))],
        out_specs=[pl.BlockSpec((_BN, D), lambda i: (i, 0)),
                   pl.BlockSpec((NC, _BN, DH), lambda i: (0, i, 0))],
        out_shape=[jax.ShapeDtypeStruct((N, D), jnp.float32),
                   jax.ShapeDtypeStruct((NC, N2, DH), jnp.float32)],
    )(x, w_self, b, w_neigh, agg, deg)


def _fuse_out_body(h_ref, ws_ref, b_ref, agg_ref, deg_ref, o_ref):
    d = jnp.maximum(deg_ref[0][:, 0:1] + deg_ref[1][:, 0:1], 1.0)
    m = jnp.concatenate([agg_ref[0], agg_ref[1]], axis=1) / d
    o_ref[...] = (jnp.dot(h_ref[...], ws_ref[...],
                          preferred_element_type=jnp.float32)
                  + m + b_ref[...])


def _tc_out(h, w_self, b, agg, deg):
    return pl.pallas_call(
        _fuse_out_body,
        grid=(_GRID,),
        in_specs=[pl.BlockSpec((_BN, D), lambda i: (i, 0)),
                  pl.BlockSpec((D, D), lambda i: (0, 0)),
                  pl.BlockSpec((1, D), lambda i: (0, 0)),
                  pl.BlockSpec((NC, _BN, DH), lambda i: (0, i, 0)),
                  pl.BlockSpec((NC, _BN, 16), lambda i: (0, i, 0))],
        out_specs=pl.BlockSpec((_BN, D), lambda i: (i, 0)),
        out_shape=jax.ShapeDtypeStruct((N, D), jnp.float32),
    )(h, w_self, b, agg, deg)


def kernel(inputs, edge_index, W_self0, W_neigh0, b0, W_self1, W_neigh1, b1):
    b0r = b0.reshape(1, D)
    b1r = b1.reshape(1, D)
    edges = edge_index.reshape(2, E // K, K)
    y0 = _tc_mm(inputs, W_neigh0)
    agg0, deg = _sc_scatter_deg(y0, edges)
    h1, y1 = _tc_mid(inputs, W_self0, b0r, W_neigh1, agg0, deg)
    agg1, = _sc_scatter(y1, edges)
    return _tc_out(h1, W_self1, b1r, agg1, deg)


# speedup vs baseline: 14.4099x; 1.2882x over previous
import functools

import jax
import jax.numpy as jnp
from jax import lax
from jax.experimental import pallas as pl
from jax.experimental.pallas import tpu as pltpu
from jax.experimental.pallas import tpu_sc as plsc

N = 10000
N2 = 10240
E = 320000
D = 128

NC = 2
NS = 16
E_TILE = E // (NC * NS)
K = 80
NCHUNK = E_TILE // K
G = 5
NSUP = NCHUNK // G
ROWS_TILE = N2 // NS
RCHUNK = K
NRC = ROWS_TILE // RCHUNK
DCHUNK = 128
NDC = ROWS_TILE // DCHUNK

_mesh = plsc.VectorSubcoreMesh(core_axis_name="c", subcore_axis_name="s")


def _sc_body(with_deg, y_hbm, edge_hbm, *refs):
    if with_deg:
        (agg_out, deg_out, src_b, dst_b, rows_v, ones_v, zdeg,
         agg_sh, deg_sh, g0, g1, g2, g3, si0, si1,
         t0, t1, t2, t3, d0, d1) = refs
    else:
        (agg_out, src_b, dst_b, rows_v, agg_sh,
         g0, g1, g2, g3, si0, si1, t0, t1, t2, t3, d0, d1) = refs
    gsem = (g0, g1, g2, g3)
    isem = (si0, si1)
    ssem = (t0, t1, t2, t3)
    dsem = (d0, d1)
    c = lax.axis_index("c")
    s = lax.axis_index("s")

    zero32 = jnp.zeros((32,), jnp.bfloat16)

    def _zrow(i, _):
        for b in range(4):
            for j in range(D // 32):
                rows_v[b, i, pl.ds(j * 32, 32)] = zero32
        return 0
    lax.fori_loop(0, K, _zrow, 0)

    rbase = s * ROWS_TILE
    for r in range(NRC):
        pltpu.async_copy(rows_v.at[r % 4],
                         agg_sh.at[pl.ds(rbase + r * RCHUNK, RCHUNK)],
                         gsem[r % 4])
    for r in range(NRC):
        pltpu.make_async_copy(rows_v.at[r % 4],
                              agg_sh.at[pl.ds(rbase + r * RCHUNK, RCHUNK)],
                              gsem[r % 4]).wait()

    if with_deg:
        zero16 = jnp.zeros((16,), jnp.float32)
        one16 = jnp.ones((16,), jnp.float32)

        def _orow(i, _):
            ones_v[i, pl.ds(0, 16)] = one16
            return 0
        lax.fori_loop(0, K, _orow, 0)

        def _zdrow(i, _):
            zdeg[i, pl.ds(0, 16)] = zero16
            return 0
        lax.fori_loop(0, DCHUNK, _zdrow, 0)

        for r in range(NDC):
            pltpu.async_copy(zdeg,
                             deg_sh.at[pl.ds(rbase + r * DCHUNK, DCHUNK)],
                             dsem[r % 2])
        for r in range(NDC):
            pltpu.make_async_copy(
                zdeg, deg_sh.at[pl.ds(rbase + r * DCHUNK, DCHUNK)],
                dsem[r % 2]).wait()
    plsc.subcore_barrier()

    cbase = (c * NS + s) * NCHUNK

    def _idx_load(u, p):
        pltpu.async_copy(edge_hbm.at[0, pl.ds(cbase + u * G, G)], src_b.at[p],
                         isem[p])
        pltpu.async_copy(edge_hbm.at[1, pl.ds(cbase + u * G, G)], dst_b.at[p],
                         isem[p])

    def _idx_wait(u, p):
        pltpu.make_async_copy(edge_hbm.at[0, pl.ds(cbase + u * G, G)],
                              src_b.at[p], isem[p]).wait()
        pltpu.make_async_copy(edge_hbm.at[1, pl.ds(cbase + u * G, G)],
                              dst_b.at[p], isem[p]).wait()

    def _gather(p, j, b):
        pltpu.async_copy(y_hbm.at[src_b.at[p, j]], rows_v.at[b], gsem[b])

    def _swait(b):
        pltpu.make_async_copy(rows_v.at[b], agg_sh.at[dst_b.at[0, 0]],
                              ssem[b]).wait()

    def _dwait(q):
        pltpu.make_async_copy(ones_v, deg_sh.at[dst_b.at[0, 0]],
                              dsem[q]).wait()

    def _consume(p, j, b, deg_wait):
        pltpu.make_async_copy(y_hbm.at[src_b.at[p, j]], rows_v.at[b],
                              gsem[b]).wait()
        pltpu.async_copy(rows_v.at[b], agg_sh.at[dst_b.at[p, j]], ssem[b],
                         add=True)
        if with_deg:
            if deg_wait:
                _dwait(j % 2)
            pltpu.async_copy(ones_v, deg_sh.at[dst_b.at[p, j]],
                             dsem[j % 2], add=True)

    def _super(u, p, bp):
        for j in range(G - 2):
            if j >= 2:
                _swait((bp + j - 2) % 4)
            _gather(p, j + 2, (bp + j + 2) % 4)
            _consume(p, j, (bp + j) % 4, deg_wait=(j >= 2))
        pn = 1 - p

        @pl.when(u + 1 < NSUP)
        def _():
            _swait((bp + G - 4) % 4)
            _idx_wait(u + 1, pn)
            _gather(pn, 0, (bp + G) % 4)
        _consume(p, G - 2, (bp + G - 2) % 4, deg_wait=True)

        @pl.when(u + 1 < NSUP)
        def _():
            _swait((bp + G - 3) % 4)
            _gather(pn, 1, (bp + G + 1) % 4)
        _consume(p, G - 1, (bp + G - 1) % 4, deg_wait=True)

        _swait((bp + G - 2) % 4)
        _swait((bp + G - 1) % 4)
        if with_deg:
            _dwait(0)
            _dwait(1)

        @pl.when(u + 2 < NSUP)
        def _():
            _idx_load(u + 2, p)

    _idx_load(0, 0)
    _idx_wait(0, 0)
    _idx_load(1, 1)
    _gather(0, 0, 0)
    _gather(0, 1, 1)

    def _quad(t, _):
        u0 = t * 4
        _super(u0, 0, 0)
        for k in range(1, 4):

            @pl.when(u0 + k < NSUP)
            def _(k=k):
                _super(u0 + k, k % 2, (G * k) % 4)
        return 0

    lax.fori_loop(0, (NSUP + 3) // 4, _quad, 0)

    _swait(1)
    _swait(2)
    plsc.subcore_barrier()

    pltpu.sync_copy(agg_sh.at[pl.ds(rbase, ROWS_TILE)],
                    agg_out.at[c, pl.ds(rbase, ROWS_TILE)])
    if with_deg:
        pltpu.sync_copy(deg_sh.at[pl.ds(rbase, ROWS_TILE)],
                        deg_out.at[c, pl.ds(rbase, ROWS_TILE)])


def _make_sc(with_deg):
    sems = [pltpu.SemaphoreType.DMA] * 12
    if with_deg:
        out_type = [jax.ShapeDtypeStruct((NC, N2, D), jnp.bfloat16),
                    jax.ShapeDtypeStruct((NC, N2, 16), jnp.float32)]
        scratch = [
            pltpu.VMEM((2, G, K), jnp.int32),
            pltpu.VMEM((2, G, K), jnp.int32),
            pltpu.VMEM((4, K, D), jnp.bfloat16),
            pltpu.VMEM((K, 16), jnp.float32),
            pltpu.VMEM((DCHUNK, 16), jnp.float32),
            pltpu.VMEM_SHARED((N2, D), jnp.bfloat16),
            pltpu.VMEM_SHARED((N2, 16), jnp.float32),
        ] + sems
    else:
        out_type = [jax.ShapeDtypeStruct((NC, N2, D), jnp.bfloat16)]
        scratch = [
            pltpu.VMEM((2, G, K), jnp.int32),
            pltpu.VMEM((2, G, K), jnp.int32),
            pltpu.VMEM((4, K, D), jnp.bfloat16),
            pltpu.VMEM_SHARED((N2, D), jnp.bfloat16),
        ] + sems
    return pl.kernel(functools.partial(_sc_body, with_deg),
                     out_type=out_type, mesh=_mesh, scratch_types=scratch,
                     compiler_params=pltpu.CompilerParams(
                         use_tc_tiling_on_sc=False),
                     name="sc_scatter_deg" if with_deg else "sc_scatter")


_sc_scatter_deg = _make_sc(True)
_sc_scatter = _make_sc(False)


_GRID = 10
_BN = 1024


def _mm_body(x_ref, w_ref, o_ref):
    y = jnp.dot(x_ref[...], w_ref[...], preferred_element_type=jnp.float32)
    o_ref[...] = y.astype(jnp.bfloat16)


def _tc_mm(x, w):
    return pl.pallas_call(
        _mm_body,
        grid=(_GRID,),
        in_specs=[pl.BlockSpec((_BN, D), lambda i: (i, 0)),
                  pl.BlockSpec((D, D), lambda i: (0, 0))],
        out_specs=pl.BlockSpec((_BN, D), lambda i: (i, 0)),
        out_shape=jax.ShapeDtypeStruct((N2, D), jnp.bfloat16),
    )(x, w)


def _neigh_mean(agg_ref, deg_ref):
    d = jnp.maximum(deg_ref[0][:, 0:1] + deg_ref[1][:, 0:1], 1.0)
    m = (agg_ref[0].astype(jnp.float32) + agg_ref[1].astype(jnp.float32))
    return m / d


def _fuse_mid_body(x_ref, ws_ref, b_ref, wn_ref, agg_ref, deg_ref,
                   h_ref, y_ref):
    h = jnp.dot(x_ref[...], ws_ref[...],
                preferred_element_type=jnp.float32)
    h = h + _neigh_mean(agg_ref, deg_ref) + b_ref[...]
    h = jnp.maximum(h, 0.0)
    h_ref[...] = h
    y = jnp.dot(h, wn_ref[...], preferred_element_type=jnp.float32)
    y_ref[...] = y.astype(jnp.bfloat16)


def _tc_mid(x, w_self, b, w_neigh, agg, deg):
    return pl.pallas_call(
        _fuse_mid_body,
        grid=(_GRID,),
        in_specs=[pl.BlockSpec((_BN, D), lambda i: (i, 0)),
                  pl.BlockSpec((D, D), lambda i: (0, 0)),
                  pl.BlockSpec((1, D), lambda i: (0, 0)),
                  pl.BlockSpec((D, D), lambda i: (0, 0)),
                  pl.BlockSpec((NC, _BN, D), lambda i: (0, i, 0)),
                  pl.BlockSpec((NC, _BN, 16), lambda i: (0, i, 0))],
        out_specs=[pl.BlockSpec((_BN, D), lambda i: (i, 0)),
                   pl.BlockSpec((_BN, D), lambda i: (i, 0))],
        out_shape=[jax.ShapeDtypeStruct((N, D), jnp.float32),
                   jax.ShapeDtypeStruct((N2, D), jnp.bfloat16)],
    )(x, w_self, b, w_neigh, agg, deg)


def _fuse_out_body(h_ref, ws_ref, b_ref, agg_ref, deg_ref, o_ref):
    o_ref[...] = (jnp.dot(h_ref[...], ws_ref[...],
                          preferred_element_type=jnp.float32)
                  + _neigh_mean(agg_ref, deg_ref) + b_ref[...])


def _tc_out(h, w_self, b, agg, deg):
    return pl.pallas_call(
        _fuse_out_body,
        grid=(_GRID,),
        in_specs=[pl.BlockSpec((_BN, D), lambda i: (i, 0)),
                  pl.BlockSpec((D, D), lambda i: (0, 0)),
                  pl.BlockSpec((1, D), lambda i: (0, 0)),
                  pl.BlockSpec((NC, _BN, D), lambda i: (0, i, 0)),
                  pl.BlockSpec((NC, _BN, 16), lambda i: (0, i, 0))],
        out_specs=pl.BlockSpec((_BN, D), lambda i: (i, 0)),
        out_shape=jax.ShapeDtypeStruct((N, D), jnp.float32),
    )(h, w_self, b, agg, deg)


def kernel(inputs, edge_index, W_self0, W_neigh0, b0, W_self1, W_neigh1, b1):
    b0r = b0.reshape(1, D)
    b1r = b1.reshape(1, D)
    edges = edge_index.reshape(2, E // K, K)
    y0 = _tc_mm(inputs, W_neigh0)
    agg0, deg = _sc_scatter_deg(y0, edges)
    h1, y1 = _tc_mid(inputs, W_self0, b0r, W_neigh1, agg0, deg)
    agg1, = _sc_scatter(y1, edges)
    return _tc_out(h1, W_self1, b1r, agg1, deg)


# speedup vs baseline: 14.4781x; 1.0047x over previous
import functools

import jax
import jax.numpy as jnp
from jax import lax
from jax.experimental import pallas as pl
from jax.experimental.pallas import tpu as pltpu
from jax.experimental.pallas import tpu_sc as plsc

N = 10000
N2 = 10240
E = 320000
D = 128

NC = 2
NS = 16
E_TILE = E // (NC * NS)
K = 80
NCHUNK = E_TILE // K
G = 5
NSUP = NCHUNK // G
ROWS_TILE = N2 // NS
RCHUNK = K
NRC = ROWS_TILE // RCHUNK
DCHUNK = 128
NDC = ROWS_TILE // DCHUNK

_mesh = plsc.VectorSubcoreMesh(core_axis_name="c", subcore_axis_name="s")


def _sc_body(with_deg, y_hbm, edge_hbm, *refs):
    if with_deg:
        (agg_out, deg_out, src_b, dst_b, rows_v, ones_v, zdeg,
         agg_sh, deg_sh, g0, g1, g2, g3, si0, si1,
         t0, t1, t2, t3, d0, d1) = refs
    else:
        (agg_out, src_b, dst_b, rows_v, agg_sh,
         g0, g1, g2, g3, si0, si1, t0, t1, t2, t3, d0, d1) = refs
    gsem = (g0, g1, g2, g3)
    isem = (si0, si1)
    ssem = (t0, t1, t2, t3)
    dsem = (d0, d1)
    c = lax.axis_index("c")
    s = lax.axis_index("s")

    zero32 = jnp.zeros((32,), jnp.bfloat16)

    def _zrow(i, _):
        for b in range(4):
            for j in range(D // 32):
                rows_v[b, i, pl.ds(j * 32, 32)] = zero32
        return 0
    lax.fori_loop(0, K, _zrow, 0)

    rbase = s * ROWS_TILE
    for r in range(NRC):
        pltpu.async_copy(rows_v.at[r % 4],
                         agg_sh.at[pl.ds(rbase + r * RCHUNK, RCHUNK)],
                         gsem[r % 4])
    for r in range(NRC):
        pltpu.make_async_copy(rows_v.at[r % 4],
                              agg_sh.at[pl.ds(rbase + r * RCHUNK, RCHUNK)],
                              gsem[r % 4]).wait()

    if with_deg:
        zero16 = jnp.zeros((16,), jnp.float32)
        one16 = jnp.ones((16,), jnp.float32)

        def _orow(i, _):
            ones_v[i, pl.ds(0, 16)] = one16
            return 0
        lax.fori_loop(0, K, _orow, 0)

        def _zdrow(i, _):
            zdeg[i, pl.ds(0, 16)] = zero16
            return 0
        lax.fori_loop(0, DCHUNK, _zdrow, 0)

        for r in range(NDC):
            pltpu.async_copy(zdeg,
                             deg_sh.at[pl.ds(rbase + r * DCHUNK, DCHUNK)],
                             dsem[r % 2])
        for r in range(NDC):
            pltpu.make_async_copy(
                zdeg, deg_sh.at[pl.ds(rbase + r * DCHUNK, DCHUNK)],
                dsem[r % 2]).wait()
    plsc.subcore_barrier()

    cbase = (c * NS + s) * NCHUNK

    def _idx_load(u, p):
        pltpu.async_copy(edge_hbm.at[0, pl.ds(cbase + u * G, G)], src_b.at[p],
                         isem[p])
        pltpu.async_copy(edge_hbm.at[1, pl.ds(cbase + u * G, G)], dst_b.at[p],
                         isem[p])

    def _idx_wait(u, p):
        pltpu.make_async_copy(edge_hbm.at[0, pl.ds(cbase + u * G, G)],
                              src_b.at[p], isem[p]).wait()
        pltpu.make_async_copy(edge_hbm.at[1, pl.ds(cbase + u * G, G)],
                              dst_b.at[p], isem[p]).wait()

    def _gather(p, j, b):
        pltpu.async_copy(y_hbm.at[src_b.at[p, j]], rows_v.at[b], gsem[b])

    def _swait(b):
        pltpu.make_async_copy(rows_v.at[b], agg_sh.at[dst_b.at[0, 0]],
                              ssem[b]).wait()

    def _dwait(q):
        pltpu.make_async_copy(ones_v, deg_sh.at[dst_b.at[0, 0]],
                              dsem[q]).wait()

    def _consume(p, j, b, deg_wait):
        pltpu.make_async_copy(y_hbm.at[src_b.at[p, j]], rows_v.at[b],
                              gsem[b]).wait()
        pltpu.async_copy(rows_v.at[b], agg_sh.at[dst_b.at[p, j]], ssem[b],
                         add=True)
        if with_deg:
            if deg_wait:
                _dwait(j % 2)
            pltpu.async_copy(ones_v, deg_sh.at[dst_b.at[p, j]],
                             dsem[j % 2], add=True)

    def _super(u, p, bp):
        for j in range(G - 2):
            if j >= 2:
                _swait((bp + j - 2) % 4)
            _gather(p, j + 2, (bp + j + 2) % 4)
            _consume(p, j, (bp + j) % 4, deg_wait=(j >= 2))
        pn = 1 - p

        @pl.when(u + 1 < NSUP)
        def _():
            _swait((bp + G - 4) % 4)
            _idx_wait(u + 1, pn)
            _gather(pn, 0, (bp + G) % 4)
        _consume(p, G - 2, (bp + G - 2) % 4, deg_wait=True)

        @pl.when(u + 1 < NSUP)
        def _():
            _swait((bp + G - 3) % 4)
            _gather(pn, 1, (bp + G + 1) % 4)
        _consume(p, G - 1, (bp + G - 1) % 4, deg_wait=True)

        _swait((bp + G - 2) % 4)
        _swait((bp + G - 1) % 4)
        if with_deg:
            _dwait(0)
            _dwait(1)

        @pl.when(u + 2 < NSUP)
        def _():
            _idx_load(u + 2, p)

    _idx_load(0, 0)
    _idx_wait(0, 0)
    _idx_load(1, 1)
    _gather(0, 0, 0)
    _gather(0, 1, 1)

    def _quad(t, _):
        u0 = t * 4
        _super(u0, 0, 0)
        for k in range(1, 4):

            @pl.when(u0 + k < NSUP)
            def _(k=k):
                _super(u0 + k, k % 2, (G * k) % 4)
        return 0

    lax.fori_loop(0, (NSUP + 3) // 4, _quad, 0)

    _swait(1)
    _swait(2)
    plsc.subcore_barrier()

    pltpu.sync_copy(agg_sh.at[pl.ds(rbase, ROWS_TILE)],
                    agg_out.at[c, pl.ds(rbase, ROWS_TILE)])
    if with_deg:
        pltpu.sync_copy(deg_sh.at[pl.ds(rbase, ROWS_TILE)],
                        deg_out.at[c, pl.ds(rbase, ROWS_TILE)])


def _make_sc(with_deg):
    sems = [pltpu.SemaphoreType.DMA] * 12
    if with_deg:
        out_type = [jax.ShapeDtypeStruct((NC, N2, D), jnp.bfloat16),
                    jax.ShapeDtypeStruct((NC, N2, 16), jnp.float32)]
        scratch = [
            pltpu.VMEM((2, G, K), jnp.int32),
            pltpu.VMEM((2, G, K), jnp.int32),
            pltpu.VMEM((4, K, D), jnp.bfloat16),
            pltpu.VMEM((K, 16), jnp.float32),
            pltpu.VMEM((DCHUNK, 16), jnp.float32),
            pltpu.VMEM_SHARED((N2, D), jnp.bfloat16),
            pltpu.VMEM_SHARED((N2, 16), jnp.float32),
        ] + sems
    else:
        out_type = [jax.ShapeDtypeStruct((NC, N2, D), jnp.bfloat16)]
        scratch = [
            pltpu.VMEM((2, G, K), jnp.int32),
            pltpu.VMEM((2, G, K), jnp.int32),
            pltpu.VMEM((4, K, D), jnp.bfloat16),
            pltpu.VMEM_SHARED((N2, D), jnp.bfloat16),
        ] + sems
    return pl.kernel(functools.partial(_sc_body, with_deg),
                     out_type=out_type, mesh=_mesh, scratch_types=scratch,
                     compiler_params=pltpu.CompilerParams(
                         use_tc_tiling_on_sc=False),
                     name="sc_scatter_deg" if with_deg else "sc_scatter")


_sc_scatter_deg = _make_sc(True)
_sc_scatter = _make_sc(False)


_GRID = 10
_BN = 1024


def _mm_body(x_ref, w_ref, o_ref):
    y = jnp.dot(x_ref[...], w_ref[...], preferred_element_type=jnp.float32)
    o_ref[...] = y.astype(jnp.bfloat16)


def _tc_mm(x, w):
    return pl.pallas_call(
        _mm_body,
        grid=(_GRID,),
        in_specs=[pl.BlockSpec((_BN, D), lambda i: (i, 0)),
                  pl.BlockSpec((D, D), lambda i: (0, 0))],
        out_specs=pl.BlockSpec((_BN, D), lambda i: (i, 0)),
        out_shape=jax.ShapeDtypeStruct((N2, D), jnp.bfloat16),
    )(x, w)


def _neigh_mean(agg_ref, deg_ref):
    d = jnp.maximum(deg_ref[0][:, 0:1] + deg_ref[1][:, 0:1], 1.0)
    m = (agg_ref[0].astype(jnp.float32) + agg_ref[1].astype(jnp.float32))
    return m / d


def _fuse_mid_body(x_ref, ws_ref, b_ref, wn_ref, agg_ref, deg_ref,
                   h_ref, y_ref):
    h = jnp.dot(x_ref[...], ws_ref[...],
                preferred_element_type=jnp.float32)
    h = h + _neigh_mean(agg_ref, deg_ref) + b_ref[...]
    h = jnp.maximum(h, 0.0)
    h_ref[...] = h.astype(jnp.bfloat16)
    y = jnp.dot(h, wn_ref[...], preferred_element_type=jnp.float32)
    y_ref[...] = y.astype(jnp.bfloat16)


def _tc_mid(x, w_self, b, w_neigh, agg, deg):
    return pl.pallas_call(
        _fuse_mid_body,
        grid=(_GRID,),
        in_specs=[pl.BlockSpec((_BN, D), lambda i: (i, 0)),
                  pl.BlockSpec((D, D), lambda i: (0, 0)),
                  pl.BlockSpec((1, D), lambda i: (0, 0)),
                  pl.BlockSpec((D, D), lambda i: (0, 0)),
                  pl.BlockSpec((NC, _BN, D), lambda i: (0, i, 0)),
                  pl.BlockSpec((NC, _BN, 16), lambda i: (0, i, 0))],
        out_specs=[pl.BlockSpec((_BN, D), lambda i: (i, 0)),
                   pl.BlockSpec((_BN, D), lambda i: (i, 0))],
        out_shape=[jax.ShapeDtypeStruct((N, D), jnp.bfloat16),
                   jax.ShapeDtypeStruct((N2, D), jnp.bfloat16)],
    )(x, w_self, b, w_neigh, agg, deg)


def _fuse_out_body(h_ref, ws_ref, b_ref, agg_ref, deg_ref, o_ref):
    o_ref[...] = (jnp.dot(h_ref[...].astype(jnp.float32), ws_ref[...],
                          preferred_element_type=jnp.float32)
                  + _neigh_mean(agg_ref, deg_ref) + b_ref[...])


def _tc_out(h, w_self, b, agg, deg):
    return pl.pallas_call(
        _fuse_out_body,
        grid=(_GRID,),
        in_specs=[pl.BlockSpec((_BN, D), lambda i: (i, 0)),
                  pl.BlockSpec((D, D), lambda i: (0, 0)),
                  pl.BlockSpec((1, D), lambda i: (0, 0)),
                  pl.BlockSpec((NC, _BN, D), lambda i: (0, i, 0)),
                  pl.BlockSpec((NC, _BN, 16), lambda i: (0, i, 0))],
        out_specs=pl.BlockSpec((_BN, D), lambda i: (i, 0)),
        out_shape=jax.ShapeDtypeStruct((N, D), jnp.float32),
    )(h, w_self, b, agg, deg)


def kernel(inputs, edge_index, W_self0, W_neigh0, b0, W_self1, W_neigh1, b1):
    b0r = b0.reshape(1, D)
    b1r = b1.reshape(1, D)
    edges = edge_index.reshape(2, E // K, K)
    y0 = _tc_mm(inputs, W_neigh0)
    agg0, deg = _sc_scatter_deg(y0, edges)
    h1, y1 = _tc_mid(inputs, W_self0, b0r, W_neigh1, agg0, deg)
    agg1, = _sc_scatter(y1, edges)
    return _tc_out(h1, W_self1, b1r, agg1, deg)
